# initial kernel scaffold (unmeasured)
import jax
import jax.numpy as jnp
from jax import lax
from jax.experimental import pallas as pl
from jax.experimental.pallas import tpu as pltpu

SCALE = 64 ** -0.5


def _compute_partials(Q3, K, V):
    b, skv, h, d = K.shape

    def body(q_ref, k_ref, v_ref, num_ref, den_ref):
        q = q_ref[0]
        k = k_ref[0]
        v = v_ref[0]
        s = lax.dot_general(
            q, k,
            dimension_numbers=(((1,), (2,)), ((0,), (1,))),
            preferred_element_type=jnp.float32,
        ) * SCALE
        p = jnp.exp(s)
        num_ref[...] = lax.dot_general(
            p, v,
            dimension_numbers=(((1,), (0,)), ((0,), (1,))),
            preferred_element_type=jnp.float32,
        )[None]
        den_ref[...] = jnp.sum(p, axis=1)[None]

    return pl.pallas_call(
        body,
        grid=(b,),
        in_specs=[
            pl.BlockSpec((1, h, d), lambda i: (i, 0, 0)),
            pl.BlockSpec((1, skv, h, d), lambda i: (i, 0, 0, 0)),
            pl.BlockSpec((1, skv, h, d), lambda i: (i, 0, 0, 0)),
        ],
        out_specs=[
            pl.BlockSpec((1, h, d), lambda i: (i, 0, 0)),
            pl.BlockSpec((1, h), lambda i: (i, 0)),
        ],
        out_shape=[
            jax.ShapeDtypeStruct((b, h, d), jnp.float32),
            jax.ShapeDtypeStruct((b, h), jnp.float32),
        ],
    )(Q3, K, V)


def _exchange_merge(num, den):
    b, h, d = num.shape

    def body(num_ref, den_ref, out_ref, nrx, drx, send_sems, recv_sems):
        my_x = lax.axis_index("x")
        my_y = lax.axis_index("y")
        nbr = (1 - my_x, my_y)

        barrier = pltpu.get_barrier_semaphore()
        pl.semaphore_signal(
            barrier, inc=1, device_id=nbr,
            device_id_type=pl.DeviceIdType.MESH,
        )
        pl.semaphore_wait(barrier, 1)

        r_num = pltpu.make_async_remote_copy(
            src_ref=num_ref, dst_ref=nrx,
            send_sem=send_sems.at[0], recv_sem=recv_sems.at[0],
            device_id=nbr, device_id_type=pl.DeviceIdType.MESH,
        )
        r_den = pltpu.make_async_remote_copy(
            src_ref=den_ref, dst_ref=drx,
            send_sem=send_sems.at[1], recv_sem=recv_sems.at[1],
            device_id=nbr, device_id_type=pl.DeviceIdType.MESH,
        )
        r_num.start()
        r_den.start()
        r_num.wait()
        r_den.wait()

        total_den = den_ref[...] + drx[...]
        out_ref[...] = (num_ref[...] + nrx[...]) / total_den[:, :, None]

    return pl.pallas_call(
        body,
        out_shape=jax.ShapeDtypeStruct((b, h, d), jnp.float32),
        in_specs=[
            pl.BlockSpec(memory_space=pltpu.VMEM),
            pl.BlockSpec(memory_space=pltpu.VMEM),
        ],
        out_specs=pl.BlockSpec(memory_space=pltpu.VMEM),
        scratch_shapes=[
            pltpu.VMEM((b, h, d), jnp.float32),
            pltpu.VMEM((b, h), jnp.float32),
            pltpu.SemaphoreType.DMA((2,)),
            pltpu.SemaphoreType.DMA((2,)),
        ],
        compiler_params=pltpu.CompilerParams(collective_id=0),
    )(num, den)


def kernel(Q, K, V):
    b, _, h, d = Q.shape
    num, den = _compute_partials(Q.reshape(b, h, d), K, V)
    out = _exchange_merge(num, den)
    return out.reshape(b, 1, h, d)


# baseline (device time: 365563 ns/iter reference)
import jax
import jax.numpy as jnp
from jax import lax
from jax.experimental import pallas as pl
from jax.experimental.pallas import tpu as pltpu

SCALE = 64 ** -0.5


def _compute_partials(Q3, K, V):
    b, skv, h, d = K.shape

    def body(q_ref, k_ref, v_ref, num_ref, den_ref):
        qa = q_ref[0]
        nums = []
        dens = []
        for hi in range(h):
            k = k_ref[0, :, hi, :]
            v = v_ref[0, :, hi, :]
            s = lax.dot_general(
                qa[hi:hi + 1, :], k,
                dimension_numbers=(((1,), (1,)), ((), ())),
                preferred_element_type=jnp.float32,
            ) * SCALE
            p = jnp.exp(s)
            nums.append(lax.dot_general(
                p, v,
                dimension_numbers=(((1,), (0,)), ((), ())),
                preferred_element_type=jnp.float32,
            ))
            dens.append(jnp.sum(p, axis=1, keepdims=True))
        num_ref[...] = jnp.concatenate(nums, axis=0)[None]
        den_ref[...] = jnp.concatenate(dens, axis=1)[None]

    return pl.pallas_call(
        body,
        grid=(b,),
        in_specs=[
            pl.BlockSpec((1, h, d), lambda i: (i, 0, 0)),
            pl.BlockSpec((1, skv, h, d), lambda i: (i, 0, 0, 0)),
            pl.BlockSpec((1, skv, h, d), lambda i: (i, 0, 0, 0)),
        ],
        out_specs=[
            pl.BlockSpec((1, h, d), lambda i: (i, 0, 0)),
            pl.BlockSpec((1, 1, h), lambda i: (i, 0, 0)),
        ],
        out_shape=[
            jax.ShapeDtypeStruct((b, h, d), jnp.float32),
            jax.ShapeDtypeStruct((b, 1, h), jnp.float32),
        ],
        compiler_params=pltpu.CompilerParams(
            vmem_limit_bytes=100 * 1024 * 1024,
        ),
    )(Q3, K, V)


def _exchange_merge(num, den):
    b, h, d = num.shape

    def body(num_ref, den_ref, out_ref, nrx, drx, send_sems, recv_sems):
        my_x = lax.axis_index("x")
        my_y = lax.axis_index("y")
        nbr = (1 - my_x, my_y)

        barrier = pltpu.get_barrier_semaphore()
        pl.semaphore_signal(
            barrier, inc=1, device_id=nbr,
            device_id_type=pl.DeviceIdType.MESH,
        )
        pl.semaphore_wait(barrier, 1)

        r_num = pltpu.make_async_remote_copy(
            src_ref=num_ref, dst_ref=nrx,
            send_sem=send_sems.at[0], recv_sem=recv_sems.at[0],
            device_id=nbr, device_id_type=pl.DeviceIdType.MESH,
        )
        r_den = pltpu.make_async_remote_copy(
            src_ref=den_ref, dst_ref=drx,
            send_sem=send_sems.at[1], recv_sem=recv_sems.at[1],
            device_id=nbr, device_id_type=pl.DeviceIdType.MESH,
        )
        r_num.start()
        r_den.start()
        r_num.wait()
        r_den.wait()

        total_den = den_ref[...] + drx[...]
        out_ref[...] = (num_ref[...] + nrx[...]) / total_den[:, 0, :, None]

    return pl.pallas_call(
        body,
        out_shape=jax.ShapeDtypeStruct((b, h, d), jnp.float32),
        in_specs=[
            pl.BlockSpec(memory_space=pltpu.VMEM),
            pl.BlockSpec(memory_space=pltpu.VMEM),
        ],
        out_specs=pl.BlockSpec(memory_space=pltpu.VMEM),
        scratch_shapes=[
            pltpu.VMEM((b, h, d), jnp.float32),
            pltpu.VMEM((b, 1, h), jnp.float32),
            pltpu.SemaphoreType.DMA((2,)),
            pltpu.SemaphoreType.DMA((2,)),
        ],
        compiler_params=pltpu.CompilerParams(collective_id=0),
    )(num, den)


def kernel(Q, K, V):
    b, _, h, d = Q.shape
    num, den = _compute_partials(Q.reshape(b, h, d), K, V)
    out = _exchange_merge(num, den)
    return out.reshape(b, 1, h, d)


# device time: 241061 ns/iter; 1.5165x vs baseline; 1.5165x over previous
import jax
import jax.numpy as jnp
from jax import lax
from jax.experimental import pallas as pl
from jax.experimental.pallas import tpu as pltpu

SCALE = 64 ** -0.5


def _compute_partials(Qf, Kf, Vf, h, d):
    b, skv, hd = Kf.shape

    def body(q_ref, k_ref, v_ref, num_ref, den_ref):
        q = q_ref[0]
        kq = k_ref[0] * q
        st = jnp.sum(kq.reshape(skv, h, d), axis=2) * SCALE
        p = jnp.exp(st)
        den_ref[...] = jnp.sum(p, axis=0)[None, None]
        pb = jnp.broadcast_to(p[:, :, None], (skv, h, d)).reshape(skv, hd)
        num_ref[...] = jnp.sum(pb * v_ref[0], axis=0)[None, None]

    return pl.pallas_call(
        body,
        grid=(b,),
        in_specs=[
            pl.BlockSpec((1, 1, hd), lambda i: (i, 0, 0)),
            pl.BlockSpec((1, skv, hd), lambda i: (i, 0, 0)),
            pl.BlockSpec((1, skv, hd), lambda i: (i, 0, 0)),
        ],
        out_specs=[
            pl.BlockSpec((1, 1, hd), lambda i: (i, 0, 0)),
            pl.BlockSpec((1, 1, h), lambda i: (i, 0, 0)),
        ],
        out_shape=[
            jax.ShapeDtypeStruct((b, 1, hd), jnp.float32),
            jax.ShapeDtypeStruct((b, 1, h), jnp.float32),
        ],
        compiler_params=pltpu.CompilerParams(
            vmem_limit_bytes=100 * 1024 * 1024,
        ),
    )(Qf, Kf, Vf)


def _exchange_merge(num, den, h, d):
    b, _, hd = num.shape

    def body(num_ref, den_ref, out_ref, nrx, drx, send_sems, recv_sems):
        my_x = lax.axis_index("x")
        my_y = lax.axis_index("y")
        nbr = (1 - my_x, my_y)

        barrier = pltpu.get_barrier_semaphore()
        pl.semaphore_signal(
            barrier, inc=1, device_id=nbr,
            device_id_type=pl.DeviceIdType.MESH,
        )
        pl.semaphore_wait(barrier, 1)

        r_num = pltpu.make_async_remote_copy(
            src_ref=num_ref, dst_ref=nrx,
            send_sem=send_sems.at[0], recv_sem=recv_sems.at[0],
            device_id=nbr, device_id_type=pl.DeviceIdType.MESH,
        )
        r_den = pltpu.make_async_remote_copy(
            src_ref=den_ref, dst_ref=drx,
            send_sem=send_sems.at[1], recv_sem=recv_sems.at[1],
            device_id=nbr, device_id_type=pl.DeviceIdType.MESH,
        )
        r_num.start()
        r_den.start()
        r_num.wait()
        r_den.wait()

        total_den = den_ref[...] + drx[...]
        inv = (1.0 / total_den)[:, :, :, None]
        scalef = jnp.broadcast_to(inv, (b, 1, h, d)).reshape(b, 1, hd)
        out_ref[...] = (num_ref[...] + nrx[...]) * scalef

    return pl.pallas_call(
        body,
        out_shape=jax.ShapeDtypeStruct((b, 1, hd), jnp.float32),
        in_specs=[
            pl.BlockSpec(memory_space=pltpu.VMEM),
            pl.BlockSpec(memory_space=pltpu.VMEM),
        ],
        out_specs=pl.BlockSpec(memory_space=pltpu.VMEM),
        scratch_shapes=[
            pltpu.VMEM((b, 1, hd), jnp.float32),
            pltpu.VMEM((b, 1, h), jnp.float32),
            pltpu.SemaphoreType.DMA((2,)),
            pltpu.SemaphoreType.DMA((2,)),
        ],
        compiler_params=pltpu.CompilerParams(collective_id=0),
    )(num, den)


def kernel(Q, K, V):
    b, _, h, d = Q.shape
    skv = K.shape[1]
    num, den = _compute_partials(
        Q.reshape(b, 1, h * d),
        K.reshape(b, skv, h * d),
        V.reshape(b, skv, h * d),
        h, d,
    )
    out = _exchange_merge(num, den, h, d)
    return out.reshape(b, 1, h, d)


# device time: 60030 ns/iter; 6.0897x vs baseline; 4.0157x over previous
import jax
import jax.numpy as jnp
from jax import lax
from jax.experimental import pallas as pl
from jax.experimental.pallas import tpu as pltpu

SCALE = 64 ** -0.5


def _compute_partials(QT, K4, V4):
    b, h, d, skv = K4.shape

    def body(qt_ref, k_ref, v_ref, num_ref, den_ref):
        qt = qt_ref[0]
        p_rows = []
        num_cols = []
        for hi in range(h):
            k = k_ref[0, hi]
            v = v_ref[0, hi]
            qcol = qt[:, hi:hi + 1]
            s = jnp.sum(k * qcol, axis=0, keepdims=True) * SCALE
            p = jnp.exp(s)
            p_rows.append(p)
            num_cols.append(jnp.sum(v * p, axis=1, keepdims=True))
        ps = jnp.concatenate(p_rows, axis=0)
        den_ref[...] = jnp.sum(ps, axis=1)[None, None]
        num_ref[...] = jnp.concatenate(num_cols, axis=1)[None]

    return pl.pallas_call(
        body,
        grid=(b,),
        in_specs=[
            pl.BlockSpec((1, d, h), lambda i: (i, 0, 0)),
            pl.BlockSpec((1, h, d, skv), lambda i: (i, 0, 0, 0)),
            pl.BlockSpec((1, h, d, skv), lambda i: (i, 0, 0, 0)),
        ],
        out_specs=[
            pl.BlockSpec((1, d, h), lambda i: (i, 0, 0)),
            pl.BlockSpec((1, 1, h), lambda i: (i, 0, 0)),
        ],
        out_shape=[
            jax.ShapeDtypeStruct((b, d, h), jnp.float32),
            jax.ShapeDtypeStruct((b, 1, h), jnp.float32),
        ],
        compiler_params=pltpu.CompilerParams(
            vmem_limit_bytes=100 * 1024 * 1024,
        ),
    )(QT, K4, V4)


def _exchange_merge(num, den):
    b, d, h = num.shape

    def body(num_ref, den_ref, out_ref, nrx, drx, send_sems, recv_sems):
        my_x = lax.axis_index("x")
        my_y = lax.axis_index("y")
        nbr = (1 - my_x, my_y)

        barrier = pltpu.get_barrier_semaphore()
        pl.semaphore_signal(
            barrier, inc=1, device_id=nbr,
            device_id_type=pl.DeviceIdType.MESH,
        )
        pl.semaphore_wait(barrier, 1)

        r_num = pltpu.make_async_remote_copy(
            src_ref=num_ref, dst_ref=nrx,
            send_sem=send_sems.at[0], recv_sem=recv_sems.at[0],
            device_id=nbr, device_id_type=pl.DeviceIdType.MESH,
        )
        r_den = pltpu.make_async_remote_copy(
            src_ref=den_ref, dst_ref=drx,
            send_sem=send_sems.at[1], recv_sem=recv_sems.at[1],
            device_id=nbr, device_id_type=pl.DeviceIdType.MESH,
        )
        r_num.start()
        r_den.start()
        r_num.wait()
        r_den.wait()

        inv = 1.0 / (den_ref[...] + drx[...])
        out_ref[...] = (num_ref[...] + nrx[...]) * inv

    return pl.pallas_call(
        body,
        out_shape=jax.ShapeDtypeStruct((b, d, h), jnp.float32),
        in_specs=[
            pl.BlockSpec(memory_space=pltpu.VMEM),
            pl.BlockSpec(memory_space=pltpu.VMEM),
        ],
        out_specs=pl.BlockSpec(memory_space=pltpu.VMEM),
        scratch_shapes=[
            pltpu.VMEM((b, d, h), jnp.float32),
            pltpu.VMEM((b, 1, h), jnp.float32),
            pltpu.SemaphoreType.DMA((2,)),
            pltpu.SemaphoreType.DMA((2,)),
        ],
        compiler_params=pltpu.CompilerParams(collective_id=0),
    )(num, den)


def kernel(Q, K, V):
    b, _, h, d = Q.shape
    K4 = jnp.transpose(K, (0, 2, 3, 1))
    V4 = jnp.transpose(V, (0, 2, 3, 1))
    QT = jnp.transpose(Q.reshape(b, h, d), (0, 2, 1))
    num, den = _compute_partials(QT, K4, V4)
    out = _exchange_merge(num, den)
    return jnp.transpose(out, (0, 2, 1)).reshape(b, 1, h, d)


# device time: 40900 ns/iter; 8.9380x vs baseline; 1.4677x over previous
import jax
import jax.numpy as jnp
from jax import lax
from jax.experimental import pallas as pl
from jax.experimental.pallas import tpu as pltpu

SCALE = 64 ** -0.5
N_Y = 2


def _compute_partials(y_idx, QT, K4, V4):
    b, h, d, skv = K4.shape
    bh = b // N_Y

    def body(y_ref, qt_ref, k_ref, v_ref, num_ref, den_ref):
        del y_ref
        qt = qt_ref[0]
        p_rows = []
        num_cols = []
        for hi in range(h):
            k = k_ref[0, hi]
            v = v_ref[0, hi]
            qcol = qt[:, hi:hi + 1]
            s = jnp.sum(k * qcol, axis=0, keepdims=True) * SCALE
            p = jnp.exp(s)
            p_rows.append(p)
            num_cols.append(jnp.sum(v * p, axis=1, keepdims=True))
        ps = jnp.concatenate(p_rows, axis=0)
        den_ref[...] = jnp.sum(ps, axis=1)[None, None]
        num_ref[...] = jnp.concatenate(num_cols, axis=1)[None]

    half = lambda i, y_ref: (y_ref[0] * bh + i, 0, 0, 0)
    half3 = lambda i, y_ref: (y_ref[0] * bh + i, 0, 0)
    return pl.pallas_call(
        body,
        grid_spec=pltpu.PrefetchScalarGridSpec(
            num_scalar_prefetch=1,
            grid=(bh,),
            in_specs=[
                pl.BlockSpec((1, d, h), half3),
                pl.BlockSpec((1, h, d, skv), half),
                pl.BlockSpec((1, h, d, skv), half),
            ],
            out_specs=[
                pl.BlockSpec((1, d, h), lambda i, y_ref: (i, 0, 0)),
                pl.BlockSpec((1, 1, h), lambda i, y_ref: (i, 0, 0)),
            ],
        ),
        out_shape=[
            jax.ShapeDtypeStruct((bh, d, h), jnp.float32),
            jax.ShapeDtypeStruct((bh, 1, h), jnp.float32),
        ],
        compiler_params=pltpu.CompilerParams(
            vmem_limit_bytes=100 * 1024 * 1024,
        ),
    )(y_idx, QT, K4, V4)


def _exchange_merge(num, den, b):
    bh, d, h = num.shape

    def body(num_ref, den_ref, out_ref, nrx, drx, ysend, yrx,
             send_sems, recv_sems):
        my_x = lax.axis_index("x")
        my_y = lax.axis_index("y")
        xnbr = (1 - my_x, my_y)
        ynbr = (my_x, 1 - my_y)

        barrier = pltpu.get_barrier_semaphore()
        for nbr in (xnbr, ynbr):
            pl.semaphore_signal(
                barrier, inc=1, device_id=nbr,
                device_id_type=pl.DeviceIdType.MESH,
            )
        pl.semaphore_wait(barrier, 2)

        r_num = pltpu.make_async_remote_copy(
            src_ref=num_ref, dst_ref=nrx,
            send_sem=send_sems.at[0], recv_sem=recv_sems.at[0],
            device_id=xnbr, device_id_type=pl.DeviceIdType.MESH,
        )
        r_den = pltpu.make_async_remote_copy(
            src_ref=den_ref, dst_ref=drx,
            send_sem=send_sems.at[1], recv_sem=recv_sems.at[1],
            device_id=xnbr, device_id_type=pl.DeviceIdType.MESH,
        )
        r_num.start()
        r_den.start()
        r_num.wait()
        r_den.wait()

        inv = 1.0 / (den_ref[...] + drx[...])
        merged = (num_ref[...] + nrx[...]) * inv

        ysend[...] = merged
        r_y = pltpu.make_async_remote_copy(
            src_ref=ysend, dst_ref=yrx,
            send_sem=send_sems.at[2], recv_sem=recv_sems.at[2],
            device_id=ynbr, device_id_type=pl.DeviceIdType.MESH,
        )
        r_y.start()
        out_ref[pl.ds(my_y * bh, bh)] = merged
        r_y.wait()
        out_ref[pl.ds((1 - my_y) * bh, bh)] = yrx[...]

    return pl.pallas_call(
        body,
        out_shape=jax.ShapeDtypeStruct((b, d, h), jnp.float32),
        in_specs=[
            pl.BlockSpec(memory_space=pltpu.VMEM),
            pl.BlockSpec(memory_space=pltpu.VMEM),
        ],
        out_specs=pl.BlockSpec(memory_space=pltpu.VMEM),
        scratch_shapes=[
            pltpu.VMEM((bh, d, h), jnp.float32),
            pltpu.VMEM((bh, 1, h), jnp.float32),
            pltpu.VMEM((bh, d, h), jnp.float32),
            pltpu.VMEM((bh, d, h), jnp.float32),
            pltpu.SemaphoreType.DMA((3,)),
            pltpu.SemaphoreType.DMA((3,)),
        ],
        compiler_params=pltpu.CompilerParams(collective_id=0),
    )(num, den)


def kernel(Q, K, V):
    b, _, h, d = Q.shape
    K4 = jnp.transpose(K, (0, 2, 3, 1))
    V4 = jnp.transpose(V, (0, 2, 3, 1))
    QT = jnp.transpose(Q.reshape(b, h, d), (0, 2, 1))
    y_idx = lax.axis_index("y").reshape(1).astype(jnp.int32)
    num, den = _compute_partials(y_idx, QT, K4, V4)
    out = _exchange_merge(num, den, b)
    return jnp.transpose(out, (0, 2, 1)).reshape(b, 1, h, d)


# device time: 40303 ns/iter; 9.0704x vs baseline; 1.0148x over previous
import jax
import jax.numpy as jnp
from jax import lax
from jax.experimental import pallas as pl
from jax.experimental.pallas import tpu as pltpu

SCALE = 64 ** -0.5
N_Y = 2


def _flash_decode(y_idx, QT, K4, V4):
    b, h, d, skv = K4.shape
    bh = b // N_Y

    def body(y_ref, qt_ref, k_ref, v_ref, out_ref,
             num_buf, den_buf, nrx, drx, yrx,
             ns_send, ns_recv, ds_send, ds_recv, y_send, y_recv):
        i = pl.program_id(0)
        my_x = lax.axis_index("x")
        my_y = lax.axis_index("y")
        xnbr = (1 - my_x, my_y)
        ynbr = (my_x, 1 - my_y)

        @pl.when(i == 0)
        def _():
            barrier = pltpu.get_barrier_semaphore()
            for nbr in (xnbr, ynbr):
                pl.semaphore_signal(
                    barrier, inc=1, device_id=nbr,
                    device_id_type=pl.DeviceIdType.MESH,
                )
            pl.semaphore_wait(barrier, 2)

        qt = qt_ref[0]
        p_rows = []
        num_cols = []
        for hi in range(h):
            k = k_ref[0, hi]
            v = v_ref[0, hi]
            qcol = qt[:, hi:hi + 1]
            s = jnp.sum(k * qcol, axis=0, keepdims=True) * SCALE
            p = jnp.exp(s)
            p_rows.append(p)
            num_cols.append(jnp.sum(v * p, axis=1, keepdims=True))
        ps = jnp.concatenate(p_rows, axis=0)
        den_buf[i] = jnp.sum(ps, axis=1)[None]
        num_buf[i] = jnp.concatenate(num_cols, axis=1)

        @pl.when(i == bh - 1)
        def _():
            r_num = pltpu.make_async_remote_copy(
                src_ref=num_buf, dst_ref=nrx,
                send_sem=ns_send, recv_sem=ns_recv,
                device_id=xnbr, device_id_type=pl.DeviceIdType.MESH,
            )
            r_den = pltpu.make_async_remote_copy(
                src_ref=den_buf, dst_ref=drx,
                send_sem=ds_send, recv_sem=ds_recv,
                device_id=xnbr, device_id_type=pl.DeviceIdType.MESH,
            )
            r_num.start()
            r_den.start()
            r_num.wait()
            r_den.wait()
            inv = 1.0 / (den_buf[...] + drx[...])
            merged = (num_buf[...] + nrx[...]) * inv
            out_ref[pl.ds(my_y * bh, bh)] = merged
            r_y = pltpu.make_async_remote_copy(
                src_ref=out_ref.at[pl.ds(my_y * bh, bh)],
                dst_ref=yrx,
                send_sem=y_send, recv_sem=y_recv,
                device_id=ynbr, device_id_type=pl.DeviceIdType.MESH,
            )
            r_y.start()
            r_y.wait()
            out_ref[pl.ds((1 - my_y) * bh, bh)] = yrx[...]

    half = lambda i, y_ref: (y_ref[0] * bh + i, 0, 0, 0)
    half3 = lambda i, y_ref: (y_ref[0] * bh + i, 0, 0)
    return pl.pallas_call(
        body,
        grid_spec=pltpu.PrefetchScalarGridSpec(
            num_scalar_prefetch=1,
            grid=(bh,),
            in_specs=[
                pl.BlockSpec((1, d, h), half3),
                pl.BlockSpec((1, h, d, skv), half),
                pl.BlockSpec((1, h, d, skv), half),
            ],
            out_specs=pl.BlockSpec(
                (b, d, h), lambda i, y_ref: (0, 0, 0)
            ),
            scratch_shapes=[
                pltpu.VMEM((bh, d, h), jnp.float32),
                pltpu.VMEM((bh, 1, h), jnp.float32),
                pltpu.VMEM((bh, d, h), jnp.float32),
                pltpu.VMEM((bh, 1, h), jnp.float32),
                pltpu.VMEM((bh, d, h), jnp.float32),
                pltpu.SemaphoreType.DMA,
                pltpu.SemaphoreType.DMA,
                pltpu.SemaphoreType.DMA,
                pltpu.SemaphoreType.DMA,
                pltpu.SemaphoreType.DMA,
                pltpu.SemaphoreType.DMA,
            ],
        ),
        out_shape=jax.ShapeDtypeStruct((b, d, h), jnp.float32),
        compiler_params=pltpu.CompilerParams(
            collective_id=0,
            vmem_limit_bytes=100 * 1024 * 1024,
        ),
    )(y_idx, QT, K4, V4)


def kernel(Q, K, V):
    b, _, h, d = Q.shape
    K4 = jnp.transpose(K, (0, 2, 3, 1))
    V4 = jnp.transpose(V, (0, 2, 3, 1))
    QT = jnp.transpose(Q.reshape(b, h, d), (0, 2, 1))
    y_idx = lax.axis_index("y").reshape(1).astype(jnp.int32)
    out = _flash_decode(y_idx, QT, K4, V4)
    return jnp.transpose(out, (0, 2, 1)).reshape(b, 1, h, d)


# device time: 38056 ns/iter; 9.6059x vs baseline; 1.0590x over previous
import jax
import jax.numpy as jnp
from jax import lax
from jax.experimental import pallas as pl
from jax.experimental.pallas import tpu as pltpu

SCALE = 64 ** -0.5
N_Y = 2


def _flash_decode(y_idx, QT, K4, V4):
    b, h, d, skv = K4.shape
    bh = b // N_Y

    def body(y_ref, qt_ref, k_ref, v_ref, out_ref,
             num_buf, den_buf, nrx, drx, yrx,
             ns_send, ns_recv, ds_send, ds_recv, y_send, y_recv):
        i = pl.program_id(0)
        my_x = lax.axis_index("x")
        my_y = lax.axis_index("y")
        xnbr = (1 - my_x, my_y)
        ynbr = (my_x, 1 - my_y)

        @pl.when(i == 0)
        def _():
            barrier = pltpu.get_barrier_semaphore()
            for nbr in (xnbr, ynbr):
                pl.semaphore_signal(
                    barrier, inc=1, device_id=nbr,
                    device_id_type=pl.DeviceIdType.MESH,
                )
            pl.semaphore_wait(barrier, 2)

        qt = qt_ref[0]
        p_rows = []
        num_cols = []
        for hi in range(h):
            k = k_ref[0, hi]
            v = v_ref[0, hi]
            qcol = qt[:, hi:hi + 1]
            s = jnp.sum(k * qcol, axis=0, keepdims=True) * SCALE
            p = jnp.exp(s)
            p_rows.append(p)
            num_cols.append(jnp.sum(v * p, axis=1, keepdims=True))
        ps = jnp.concatenate(p_rows, axis=0)
        den_row = jnp.sum(ps, axis=1)[None]
        num_rows = jnp.concatenate(num_cols, axis=1)
        num_buf[i] = jnp.concatenate([num_rows, den_row], axis=0)

        def head_rdma():
            return pltpu.make_async_remote_copy(
                src_ref=num_buf.at[pl.ds(0, bh - 1)],
                dst_ref=nrx.at[pl.ds(0, bh - 1)],
                send_sem=ns_send, recv_sem=ns_recv,
                device_id=xnbr, device_id_type=pl.DeviceIdType.MESH,
            )

        @pl.when(i == bh - 2)
        def _():
            head_rdma().start()

        @pl.when(i == bh - 1)
        def _():
            r_tail = pltpu.make_async_remote_copy(
                src_ref=num_buf.at[pl.ds(bh - 1, 1)],
                dst_ref=nrx.at[pl.ds(bh - 1, 1)],
                send_sem=ds_send, recv_sem=ds_recv,
                device_id=xnbr, device_id_type=pl.DeviceIdType.MESH,
            )
            r_tail.start()
            head_rdma().wait()
            r_tail.wait()
            both = num_buf[...] + nrx[...]
            inv = 1.0 / both[:, d:d + 1, :]
            merged = both[:, :d, :] * inv
            out_ref[pl.ds(my_y * bh, bh)] = merged
            r_y = pltpu.make_async_remote_copy(
                src_ref=out_ref.at[pl.ds(my_y * bh, bh)],
                dst_ref=yrx,
                send_sem=y_send, recv_sem=y_recv,
                device_id=ynbr, device_id_type=pl.DeviceIdType.MESH,
            )
            r_y.start()
            r_y.wait()
            out_ref[pl.ds((1 - my_y) * bh, bh)] = yrx[...]

    half = lambda i, y_ref: (y_ref[0] * bh + i, 0, 0, 0)
    half3 = lambda i, y_ref: (y_ref[0] * bh + i, 0, 0)
    return pl.pallas_call(
        body,
        grid_spec=pltpu.PrefetchScalarGridSpec(
            num_scalar_prefetch=1,
            grid=(bh,),
            in_specs=[
                pl.BlockSpec((1, d, h), half3),
                pl.BlockSpec((1, h, d, skv), half),
                pl.BlockSpec((1, h, d, skv), half),
            ],
            out_specs=pl.BlockSpec(
                (b, d, h), lambda i, y_ref: (0, 0, 0)
            ),
            scratch_shapes=[
                pltpu.VMEM((bh, d + 1, h), jnp.float32),
                pltpu.VMEM((bh, 1, h), jnp.float32),
                pltpu.VMEM((bh, d + 1, h), jnp.float32),
                pltpu.VMEM((bh, 1, h), jnp.float32),
                pltpu.VMEM((bh, d, h), jnp.float32),
                pltpu.SemaphoreType.DMA,
                pltpu.SemaphoreType.DMA,
                pltpu.SemaphoreType.DMA,
                pltpu.SemaphoreType.DMA,
                pltpu.SemaphoreType.DMA,
                pltpu.SemaphoreType.DMA,
            ],
        ),
        out_shape=jax.ShapeDtypeStruct((b, d, h), jnp.float32),
        compiler_params=pltpu.CompilerParams(
            collective_id=0,
            vmem_limit_bytes=100 * 1024 * 1024,
        ),
    )(y_idx, QT, K4, V4)


def kernel(Q, K, V):
    b, _, h, d = Q.shape
    K4 = jnp.transpose(K, (0, 2, 3, 1))
    V4 = jnp.transpose(V, (0, 2, 3, 1))
    QT = jnp.transpose(Q.reshape(b, h, d), (0, 2, 1))
    y_idx = lax.axis_index("y").reshape(1).astype(jnp.int32)
    out = _flash_decode(y_idx, QT, K4, V4)
    return jnp.transpose(out, (0, 2, 1)).reshape(b, 1, h, d)


# device time: 33577 ns/iter; 10.8873x vs baseline; 1.1334x over previous
import jax
import jax.numpy as jnp
from jax import lax
from jax.experimental import pallas as pl
from jax.experimental.pallas import tpu as pltpu

SCALE = 64 ** -0.5
N_Y = 2


def _flash_decode(y_idx, QT, K4, V4):
    b, h, d, skv = K4.shape
    bh = b // N_Y

    def body(y_ref, qt_ref, k_ref, v_ref, out_ref,
             num_buf, den_buf, nrx, drx, yrx,
             ns_send, ns_recv, ds_send, ds_recv, y_send, y_recv):
        i = pl.program_id(0)
        my_x = lax.axis_index("x")
        my_y = lax.axis_index("y")
        xnbr = (1 - my_x, my_y)
        ynbr = (my_x, 1 - my_y)

        @pl.when(i == 0)
        def _():
            barrier = pltpu.get_barrier_semaphore()
            for nbr in (xnbr, ynbr):
                pl.semaphore_signal(
                    barrier, inc=1, device_id=nbr,
                    device_id_type=pl.DeviceIdType.MESH,
                )
            pl.semaphore_wait(barrier, 2)

        qt = jnp.transpose(qt_ref[0], (1, 0))
        p_rows = []
        num_cols = []
        for hi in range(h):
            k = k_ref[0, hi]
            v = v_ref[0, hi]
            qcol = qt[:, hi:hi + 1]
            s = jnp.sum(k * qcol, axis=0, keepdims=True) * SCALE
            p = jnp.exp(s)
            p_rows.append(p)
            num_cols.append(jnp.sum(v * p, axis=1, keepdims=True))
        ps = jnp.concatenate(p_rows, axis=0)
        den_row = jnp.sum(ps, axis=1)[None]
        num_rows = jnp.concatenate(num_cols, axis=1)
        num_buf[i] = jnp.concatenate([num_rows, den_row], axis=0)

        def head_rdma():
            return pltpu.make_async_remote_copy(
                src_ref=num_buf.at[pl.ds(0, bh - 1)],
                dst_ref=nrx.at[pl.ds(0, bh - 1)],
                send_sem=ns_send, recv_sem=ns_recv,
                device_id=xnbr, device_id_type=pl.DeviceIdType.MESH,
            )

        @pl.when(i == bh - 2)
        def _():
            head_rdma().start()

        @pl.when(i == bh - 1)
        def _():
            r_tail = pltpu.make_async_remote_copy(
                src_ref=num_buf.at[pl.ds(bh - 1, 1)],
                dst_ref=nrx.at[pl.ds(bh - 1, 1)],
                send_sem=ds_send, recv_sem=ds_recv,
                device_id=xnbr, device_id_type=pl.DeviceIdType.MESH,
            )
            r_tail.start()
            head_rdma().wait()
            r_tail.wait()
            both = num_buf[...] + nrx[...]
            inv = 1.0 / both[:, d:d + 1, :]
            merged = jnp.transpose(both[:, :d, :] * inv, (0, 2, 1))
            out_ref[pl.ds(my_y * bh, bh)] = merged
            r_y = pltpu.make_async_remote_copy(
                src_ref=out_ref.at[pl.ds(my_y * bh, bh)],
                dst_ref=yrx,
                send_sem=y_send, recv_sem=y_recv,
                device_id=ynbr, device_id_type=pl.DeviceIdType.MESH,
            )
            r_y.start()
            r_y.wait()
            out_ref[pl.ds((1 - my_y) * bh, bh)] = yrx[...]

    half = lambda i, y_ref: (y_ref[0] * bh + i, 0, 0, 0)
    half3 = lambda i, y_ref: (y_ref[0] * bh + i, 0, 0)
    return pl.pallas_call(
        body,
        grid_spec=pltpu.PrefetchScalarGridSpec(
            num_scalar_prefetch=1,
            grid=(bh,),
            in_specs=[
                pl.BlockSpec((1, h, d), half3),
                pl.BlockSpec((1, h, d, skv), half),
                pl.BlockSpec((1, h, d, skv), half),
            ],
            out_specs=pl.BlockSpec(
                (b, h, d), lambda i, y_ref: (0, 0, 0)
            ),
            scratch_shapes=[
                pltpu.VMEM((bh, d + 1, h), jnp.float32),
                pltpu.VMEM((bh, 1, h), jnp.float32),
                pltpu.VMEM((bh, d + 1, h), jnp.float32),
                pltpu.VMEM((bh, 1, h), jnp.float32),
                pltpu.VMEM((bh, h, d), jnp.float32),
                pltpu.SemaphoreType.DMA,
                pltpu.SemaphoreType.DMA,
                pltpu.SemaphoreType.DMA,
                pltpu.SemaphoreType.DMA,
                pltpu.SemaphoreType.DMA,
                pltpu.SemaphoreType.DMA,
            ],
        ),
        out_shape=jax.ShapeDtypeStruct((b, h, d), jnp.float32),
        compiler_params=pltpu.CompilerParams(
            collective_id=0,
            vmem_limit_bytes=100 * 1024 * 1024,
        ),
    )(y_idx, QT, K4, V4)


def kernel(Q, K, V):
    b, _, h, d = Q.shape
    K4 = jnp.transpose(K, (0, 2, 3, 1))
    V4 = jnp.transpose(V, (0, 2, 3, 1))
    y_idx = lax.axis_index("y").reshape(1).astype(jnp.int32)
    out = _flash_decode(y_idx, Q.reshape(b, h, d), K4, V4)
    return out.reshape(b, 1, h, d)
